# Initial kernel scaffold; baseline (speedup 1.0000x reference)
import jax
import jax.numpy as jnp
from jax.experimental import pallas as pl


def kernel(x, pos, edge_attr, node_sigma, edge_index, params):
    raise NotImplementedError("write your pallas kernel here")


# speedup vs baseline: 57.8437x; 57.8437x over previous
import functools
import math

import jax
import jax.numpy as jnp
from jax import lax
from jax.experimental import pallas as pl
from jax.experimental.pallas import tpu as pltpu
from jax.experimental.pallas import tpu_sc as plsc

N = 10000
E = 160000
SIGMA_MIN = 0.01 * math.pi
SIGMA_MAX = math.pi
_LAYERS = [(32, 96), (96, 160), (160, 192), (192, 192)]

NTILES = 32
EPT = E // NTILES
CHUNK = 40
NCHUNK = EPT // CHUNK
NCHUNK_PAD = 128
NP = 10240
ROWS_PT = NP // 16

_SMEAR_STEP = 5.0 / 49.0
_SMEAR_COEFF = -0.5 / (_SMEAR_STEP * _SMEAR_STEP)
_C3 = math.sqrt(3.0)
_C5 = math.sqrt(5.0)
_C15 = math.sqrt(15.0)


def _mesh():
    return plsc.VectorSubcoreMesh(core_axis_name="c", subcore_axis_name="s",
                                  num_cores=2, num_subcores=16)


@functools.lru_cache(maxsize=None)
def _gather2_kernel(DA, DB):
    @functools.partial(
        pl.kernel,
        out_type=(jax.ShapeDtypeStruct((E, DA), jnp.float32),
                  jax.ShapeDtypeStruct((E, DB), jnp.float32)),
        mesh=_mesh(),
        scratch_types=[
            pltpu.VMEM((NCHUNK_PAD, CHUNK), jnp.int32),
            pltpu.VMEM((NCHUNK_PAD, CHUNK), jnp.int32),
            pltpu.VMEM((CHUNK, DA), jnp.float32),
            pltpu.VMEM((CHUNK, DB), jnp.float32),
            pltpu.SemaphoreType.DMA,
            pltpu.SemaphoreType.DMA,
        ],
        compiler_params=pltpu.CompilerParams(use_tc_tiling_on_sc=False),
    )
    def gk(tab_a, idx_a, tab_b, idx_b, out_a, out_b,
           ia, ib, buf_a, buf_b, sem_a, sem_b):
        t = lax.axis_index("c") * 16 + lax.axis_index("s")
        pltpu.sync_copy(idx_a.at[t], ia)
        pltpu.sync_copy(idx_b.at[t], ib)

        def body(j, carry):
            base = t * EPT + j * CHUNK
            cp_a = pltpu.async_copy(tab_a.at[ia.at[j]], buf_a, sem_a)
            cp_b = pltpu.async_copy(tab_b.at[ib.at[j]], buf_b, sem_b)
            cp_a.wait()
            pltpu.sync_copy(buf_a, out_a.at[pl.ds(base, CHUNK)])
            cp_b.wait()
            pltpu.sync_copy(buf_b, out_b.at[pl.ds(base, CHUNK)])
            return carry

        lax.fori_loop(0, NCHUNK, body, 0)

    return gk


_SPMEM_ACC_WORDS = 1_800_000


@functools.lru_cache(maxsize=None)
def _scatter_kernel(DF, C0, DC):
    @functools.partial(
        pl.kernel,
        out_type=jax.ShapeDtypeStruct((2, NP, DC), jnp.float32),
        mesh=_mesh(),
        scratch_types=[
            pltpu.VMEM((NCHUNK_PAD, CHUNK), jnp.int32),
            pltpu.VMEM((CHUNK, DC), jnp.float32),
            pltpu.VMEM_SHARED((NP, DC), jnp.float32),
        ],
        compiler_params=pltpu.CompilerParams(use_tc_tiling_on_sc=False),
    )
    def sk(vals, idx3, zeros, out, iv, buf, acc):
        c = lax.axis_index("c")
        s = lax.axis_index("s")
        t = c * 16 + s
        r0 = s * ROWS_PT
        pltpu.sync_copy(zeros.at[pl.ds(r0, ROWS_PT)],
                        acc.at[pl.ds(r0, ROWS_PT)])
        pltpu.sync_copy(idx3.at[t], iv)
        plsc.subcore_barrier()

        def body(j, carry):
            base = t * EPT + j * CHUNK
            pltpu.sync_copy(vals.at[pl.ds(base, CHUNK), pl.ds(C0, DC)], buf)
            pltpu.sync_copy(buf, acc.at[iv.at[j]], add=True)
            return carry

        lax.fori_loop(0, NCHUNK, body, 0)
        plsc.subcore_barrier()
        pltpu.sync_copy(acc.at[pl.ds(r0, ROWS_PT)],
                        out.at[c, pl.ds(r0, ROWS_PT)])

    return sk


def _run_gather2(tab_a, idx_a3, tab_b, idx_b3):
    return _gather2_kernel(tab_a.shape[1], tab_b.shape[1])(
        tab_a, idx_a3, tab_b, idx_b3)


def _run_scatter(vals, idx3):
    D = vals.shape[1]
    if NP * D <= _SPMEM_ACC_WORDS:
        parts = [(0, D)]
    else:
        h = ((D // 2) + 7) // 8 * 8
        parts = [(0, h), (h, D - h)]
    outs = [
        _scatter_kernel(D, c0, dc)(vals, idx3, jnp.zeros((NP, dc), jnp.float32))
        [:, :N, :]
        for c0, dc in parts
    ]
    return outs[0] if len(outs) == 1 else jnp.concatenate(outs, axis=2)


_BN_NODE = 2000
_BE = 2000


def _node_body(sig_ref, x_ref, pos8_ref, freq_ref, w1x_ref, w1s_ref, b1_ref,
               w2_ref, b2_ref, tab40_ref, na0_ref):
    tlog = jnp.log(sig_ref[...] * (1.0 / SIGMA_MIN)) * (
        10000.0 / math.log(SIGMA_MAX / SIGMA_MIN))
    e = tlog * freq_ref[...]
    semb = jnp.concatenate([jnp.sin(e), jnp.cos(e)], axis=1)
    h = jnp.maximum(
        jnp.dot(x_ref[...], w1x_ref[...], preferred_element_type=jnp.float32)
        + jnp.dot(semb, w1s_ref[...], preferred_element_type=jnp.float32)
        + b1_ref[...], 0.0)
    na0 = jnp.dot(h, w2_ref[...], preferred_element_type=jnp.float32) + b2_ref[...]
    tab40_ref[...] = jnp.concatenate([semb, pos8_ref[...]], axis=1)
    na0_ref[...] = na0


def _node_call(sig2, x, pos8, freq, w1x, w1s, b1, w2, b2):
    g = N // _BN_NODE
    row = lambda i: (i, 0)
    full = lambda i: (0, 0)
    return pl.pallas_call(
        _node_body,
        grid=(g,),
        in_specs=[
            pl.BlockSpec((_BN_NODE, 1), row),
            pl.BlockSpec((_BN_NODE, 74), row),
            pl.BlockSpec((_BN_NODE, 8), row),
            pl.BlockSpec((1, 16), full),
            pl.BlockSpec((74, 32), full),
            pl.BlockSpec((32, 32), full),
            pl.BlockSpec((1, 32), full),
            pl.BlockSpec((32, 32), full),
            pl.BlockSpec((1, 32), full),
        ],
        out_specs=[pl.BlockSpec((_BN_NODE, 40), row),
                   pl.BlockSpec((_BN_NODE, 32), row)],
        out_shape=[jax.ShapeDtypeStruct((N, 40), jnp.float32),
                   jax.ShapeDtypeStruct((N, 32), jnp.float32)],
    )(sig2, x, pos8, freq, w1x, w1s, b1, w2, b2)


def _edge_body(eattr_ref, g40_ref, posd_ref, w1a_ref, w1s_ref, w1r_ref,
               b1_ref, w2_ref, b2_ref, off_ref, es_ref):
    g40 = g40_ref[...]
    semb = g40[:, 0:32]
    ps = g40[:, 32:35]
    pd = posd_ref[:, 0:3]
    v = pd - ps
    r = jnp.sqrt(jnp.sum(v * v, axis=1, keepdims=True))
    d = r - off_ref[...]
    lemb = jnp.exp(_SMEAR_COEFF * d * d)
    u = v / (r + 1e-9)
    ux = u[:, 0:1]
    uy = u[:, 1:2]
    uz = u[:, 2:3]
    sh = jnp.concatenate([
        jnp.ones_like(ux),
        _C3 * uy, _C3 * uz, _C3 * ux,
        _C15 * ux * uy, _C15 * uy * uz,
        (_C5 / 2.0) * (3.0 * uz * uz - 1.0),
        _C15 * ux * uz, (_C15 / 2.0) * (ux * ux - uy * uy),
    ], axis=1)
    h = jnp.maximum(
        jnp.dot(eattr_ref[...], w1a_ref[...], preferred_element_type=jnp.float32)
        + jnp.dot(semb, w1s_ref[...], preferred_element_type=jnp.float32)
        + jnp.dot(lemb, w1r_ref[...], preferred_element_type=jnp.float32)
        + b1_ref[...], 0.0)
    ea = jnp.dot(h, w2_ref[...], preferred_element_type=jnp.float32) + b2_ref[...]
    pad = jnp.zeros((ea.shape[0], 7), jnp.float32)
    es_ref[...] = jnp.concatenate([ea, sh, pad], axis=1)


def _edge_call(edge_attr, g40, posd, w1a, w1s, w1r, b1, w2, b2, offs):
    g = E // _BE
    row = lambda i: (i, 0)
    full = lambda i: (0, 0)
    return pl.pallas_call(
        _edge_body,
        grid=(g,),
        in_specs=[
            pl.BlockSpec((_BE, 4), row),
            pl.BlockSpec((_BE, 40), row),
            pl.BlockSpec((_BE, 8), row),
            pl.BlockSpec((4, 8), full),
            pl.BlockSpec((32, 8), full),
            pl.BlockSpec((50, 8), full),
            pl.BlockSpec((1, 8), full),
            pl.BlockSpec((8, 8), full),
            pl.BlockSpec((1, 8), full),
            pl.BlockSpec((1, 50), full),
        ],
        out_specs=pl.BlockSpec((_BE, 24), row),
        out_shape=jax.ShapeDtypeStruct((E, 24), jnp.float32),
    )(edge_attr, g40, posd, w1a, w1s, w1r, b1, w2, b2, offs)


def _dense_body(es_ref, nsrc_ref, ndst_ref, w1a_ref, w1s_ref, w1d_ref,
                b1_ref, w2_ref, b2_ref, tpw_ref, shv_ref, out_ref):
    es = es_ref[...]
    ea = es[:, 0:8]
    sh = es[:, 8:17]
    nsrc = nsrc_ref[...]
    s32 = nsrc[:, 0:32]
    d32 = ndst_ref[...]
    h = jnp.maximum(
        jnp.dot(ea, w1a_ref[...], preferred_element_type=jnp.float32)
        + jnp.dot(s32, w1s_ref[...], preferred_element_type=jnp.float32)
        + jnp.dot(d32, w1d_ref[...], preferred_element_type=jnp.float32)
        + b1_ref[...], 0.0)
    gate = jnp.dot(h, w2_ref[...], preferred_element_type=jnp.float32) + b2_ref[...]
    tp = jnp.dot(nsrc, tpw_ref[...], preferred_element_type=jnp.float32)
    shg = jnp.dot(sh, shv_ref[...], preferred_element_type=jnp.float32)
    out_ref[...] = tp * gate * shg


def _dense_call(es, nsrc, ndst, w1a, w1s, w1d, b1, w2, b2, tpw, shv, di, do):
    g = E // _BE
    row = lambda i: (i, 0)
    full = lambda i: (0, 0)
    return pl.pallas_call(
        _dense_body,
        grid=(g,),
        in_specs=[
            pl.BlockSpec((_BE, 24), row),
            pl.BlockSpec((_BE, di), row),
            pl.BlockSpec((_BE, 32), row),
            pl.BlockSpec((8, 72), full),
            pl.BlockSpec((32, 72), full),
            pl.BlockSpec((32, 72), full),
            pl.BlockSpec((1, 72), full),
            pl.BlockSpec((72, do), full),
            pl.BlockSpec((1, do), full),
            pl.BlockSpec((di, do), full),
            pl.BlockSpec((9, do), full),
        ],
        out_specs=pl.BlockSpec((_BE, do), row),
        out_shape=jax.ShapeDtypeStruct((E, do), jnp.float32),
    )(es, nsrc, ndst, w1a, w1s, w1d, b1, w2, b2, tpw, shv)


def _bn_stats_body(sums_ref, deg_ref, nap_ref, out_ref, stats_ref, acc_ref):
    i = pl.program_id(0)

    @pl.when(i == 0)
    def _():
        acc_ref[...] = jnp.zeros_like(acc_ref)

    blk = sums_ref[0] + sums_ref[1]
    deg = deg_ref[0][:, 0:1] + deg_ref[1][:, 0:1]
    invd = 1.0 / jnp.maximum(deg, 1.0)
    nap = nap_ref[...]
    do = blk.shape[1]
    di = nap.shape[1]
    if di < do:
        nap = jnp.concatenate(
            [nap, jnp.zeros((nap.shape[0], do - di), jnp.float32)], axis=1)
    outb = blk * invd + nap
    out_ref[...] = outb
    acc_ref[0:1] = acc_ref[0:1] + jnp.sum(outb, axis=0, keepdims=True)
    acc_ref[1:2] = acc_ref[1:2] + jnp.sum(outb * outb, axis=0, keepdims=True)

    @pl.when(i == pl.num_programs(0) - 1)
    def _():
        mean = acc_ref[0:1] * (1.0 / N)
        var = acc_ref[1:2] * (1.0 / N) - mean * mean
        stats_ref[0:1] = mean
        stats_ref[1:2] = lax.rsqrt(var + 1e-5)


def _bn_stats_call(sums, deg2, na_prev, do):
    g = N // _BN_NODE
    di = na_prev.shape[1]
    return pl.pallas_call(
        _bn_stats_body,
        grid=(g,),
        in_specs=[
            pl.BlockSpec((2, _BN_NODE, do), lambda i: (0, i, 0)),
            pl.BlockSpec((2, _BN_NODE, 8), lambda i: (0, i, 0)),
            pl.BlockSpec((_BN_NODE, di), lambda i: (i, 0)),
        ],
        out_specs=[pl.BlockSpec((_BN_NODE, do), lambda i: (i, 0)),
                   pl.BlockSpec((8, do), lambda i: (0, 0))],
        out_shape=[jax.ShapeDtypeStruct((N, do), jnp.float32),
                   jax.ShapeDtypeStruct((8, do), jnp.float32)],
        scratch_shapes=[pltpu.VMEM((8, do), jnp.float32)],
    )(sums, deg2, na_prev)


def _bn_apply_body(out_ref, stats_ref, g_ref, b_ref, na_ref, na32_ref):
    mean = stats_ref[0:1]
    rstd = stats_ref[1:2]
    na = (out_ref[...] - mean) * rstd * g_ref[...] + b_ref[...]
    na_ref[...] = na
    na32_ref[...] = na[:, 0:32]


def _bn_apply_call(out, stats, bn_g, bn_b, do):
    g = N // _BN_NODE
    row = lambda i: (i, 0)
    full = lambda i: (0, 0)
    return pl.pallas_call(
        _bn_apply_body,
        grid=(g,),
        in_specs=[
            pl.BlockSpec((_BN_NODE, do), row),
            pl.BlockSpec((8, do), full),
            pl.BlockSpec((1, do), full),
            pl.BlockSpec((1, do), full),
        ],
        out_specs=[pl.BlockSpec((_BN_NODE, do), row),
                   pl.BlockSpec((_BN_NODE, 32), row)],
        out_shape=[jax.ShapeDtypeStruct((N, do), jnp.float32),
                   jax.ShapeDtypeStruct((N, 32), jnp.float32)],
    )(out, stats, bn_g, bn_b)


def kernel(x, pos, edge_attr, node_sigma, edge_index, params):
    ei = edge_index.astype(jnp.int32)
    pad_rows = ((0, 0), (0, NCHUNK_PAD - NCHUNK), (0, 0))
    src3 = jnp.pad(ei[0].reshape(NTILES, NCHUNK, CHUNK), pad_rows)
    dst3 = jnp.pad(ei[1].reshape(NTILES, NCHUNK, CHUNK), pad_rows)
    pos8 = jnp.pad(pos, ((0, 0), (0, 5)))
    sig2 = node_sigma.reshape(N, 1)
    p = params

    freq = jnp.exp(jnp.arange(16, dtype=jnp.float32)
                   * (-math.log(10000.0) / 15.0)).reshape(1, 16)
    tab40, na = _node_call(
        sig2, x, pos8, freq,
        p['ne_W1'][:74], p['ne_W1'][74:],
        p['ne_b1'].reshape(1, -1), p['ne_W2'], p['ne_b2'].reshape(1, -1))

    g40, posd = _run_gather2(tab40, src3, pos8, dst3)

    offs = jnp.linspace(0.0, 5.0, 50).astype(jnp.float32).reshape(1, 50)
    es = _edge_call(
        edge_attr, g40, posd,
        p['ee_W1'][:4], p['ee_W1'][4:36], p['ee_W1'][36:],
        p['ee_b1'].reshape(1, -1), p['ee_W2'], p['ee_b2'].reshape(1, -1), offs)

    deg2 = _run_scatter(jnp.ones((E, 8), jnp.float32), dst3)

    na32 = na
    for l, (di, do) in enumerate(_LAYERS):
        lp = p['layer%d' % l]
        nsrc, ndst = _run_gather2(na, src3, na32, dst3)
        summand = _dense_call(
            es, nsrc, ndst,
            lp['fc_W1'][:8], lp['fc_W1'][8:40], lp['fc_W1'][40:],
            lp['fc_b1'].reshape(1, -1), lp['fc_W2'], lp['fc_b2'].reshape(1, -1),
            lp['tp_W'], lp['sh_V'], di, do)
        sums = _run_scatter(summand, dst3)
        out, stats = _bn_stats_call(sums, deg2, na, do)
        na, na32 = _bn_apply_call(
            out, stats, lp['bn_g'].reshape(1, -1), lp['bn_b'].reshape(1, -1), do)

    return (na, edge_index)
